# baseline (speedup 1.0000x reference)
import jax
import jax.numpy as jnp
from jax.experimental import pallas as pl

_N = 1024
_FEAT = 8
_HID = 32
_LAT = 16
_DH = 2 * _HID
_RBLK = 128
_TBLK = 128


def _encoder_kernel(a_ref, x_ref, w1_ref, b1_ref, w2_ref, b2_ref, w3_ref, b3_ref,
                    wmu_ref, bmu_ref, wlv_ref, blv_ref, we1a_ref, we1b_ref, be1_ref,
                    mu_ref, lv_ref, pt_ref, qt_ref):
    f32 = jnp.float32
    a = a_ref[...]
    ii = jax.lax.broadcasted_iota(jnp.int32, (_N, _N), 0)
    jj = jax.lax.broadcasted_iota(jnp.int32, (_N, _N), 1)
    eye = ii == jj
    d_col = jnp.sum(jnp.where(eye, a, 0.0), axis=1, keepdims=True)
    a_hat = a + jnp.where(eye & (d_col == 0.0), 1.0, 0.0)
    ones = jnp.ones((_N, 1), f32)
    deg = jax.lax.dot_general(a_hat, ones, (((0,), (0,)), ((), ())),
                              preferred_element_type=f32)
    dis = jnp.where(deg > 0.0, jax.lax.rsqrt(deg), 0.0)

    def agg(v):
        return dis * jax.lax.dot_general(a_hat, dis * v, (((0,), (0,)), ((), ())),
                                         preferred_element_type=f32)

    x = x_ref[...]
    h1 = jnp.maximum(agg(jnp.dot(x, w1_ref[...], preferred_element_type=f32)) + b1_ref[...], 0.0)
    h2 = jnp.maximum(agg(jnp.dot(h1, w2_ref[...], preferred_element_type=f32)) + b2_ref[...], 0.0) + h1
    h3 = jnp.maximum(agg(jnp.dot(h2, w3_ref[...], preferred_element_type=f32)) + b3_ref[...], 0.0) + h2
    mu = jnp.dot(h3, wmu_ref[...], preferred_element_type=f32) + bmu_ref[...]
    mu_ref[...] = mu
    lv_ref[...] = jnp.dot(h3, wlv_ref[...], preferred_element_type=f32) + blv_ref[...]
    pt_ref[...] = jax.lax.dot_general(we1a_ref[...], mu, (((0,), (1,)), ((), ())),
                                      preferred_element_type=f32) + be1_ref[...]
    qt_ref[...] = jax.lax.dot_general(we1b_ref[...], mu, (((0,), (1,)), ((), ())),
                                      preferred_element_type=f32)


def _decoder_kernel(ptb_ref, qtb_ref, ptc_ref, qtc_ref, w2bd_ref, be2_ref,
                    we3_ref, be3_ref, out_ref):
    tt = pl.program_id(0)
    cc = pl.program_id(1)
    ptc = ptc_ref[...]
    qtc = qtc_ref[...]
    w2bd = w2bd_ref[...]
    be2 = be2_ref[...]
    we3 = we3_ref[...]
    be3 = be3_ref[0, 0]

    def tail(hu1, hu2):
        hh = jnp.concatenate([hu1, hu2], axis=0)
        g = jnp.maximum(jnp.dot(w2bd, hh, preferred_element_type=jnp.float32) + be2, 0.0)
        t = g * we3
        s1 = jnp.sum(t[0:_HID], axis=0, keepdims=True) + be3
        s2 = jnp.sum(t[_HID:], axis=0, keepdims=True) + be3
        return jax.nn.sigmoid(jnp.concatenate([s1, s2], axis=0))

    @pl.when(cc > tt)
    def _():
        for s in range(_RBLK // 2):
            c = 2 * s
            hu1 = jnp.maximum(ptb_ref[:, c:c + 1] + qtc, 0.0)
            hu2 = jnp.maximum(ptb_ref[:, c + 1:c + 2] + qtc, 0.0)
            out_ref[c:c + 2, :] = tail(hu1, hu2)

    @pl.when(cc < tt)
    def _():
        for s in range(_RBLK // 2):
            c = 2 * s
            hu1 = jnp.maximum(ptc + qtb_ref[:, c:c + 1], 0.0)
            hu2 = jnp.maximum(ptc + qtb_ref[:, c + 1:c + 2], 0.0)
            out_ref[c:c + 2, :] = tail(hu1, hu2)

    @pl.when(cc == tt)
    def _():
        lane = jax.lax.broadcasted_iota(jnp.int32, (1, _TBLK), 1)
        lane2 = jax.lax.broadcasted_iota(jnp.int32, (2, _TBLK), 1)
        row2 = jax.lax.broadcasted_iota(jnp.int32, (2, _TBLK), 0)
        for s in range(_RBLK // 2):
            c = 2 * s
            pc1 = ptb_ref[:, c:c + 1]
            qc1 = qtb_ref[:, c:c + 1]
            pc2 = ptb_ref[:, c + 1:c + 2]
            qc2 = qtb_ref[:, c + 1:c + 2]
            hu1 = jnp.maximum(jnp.where(lane > c, pc1 + qtc, ptc + qc1), 0.0)
            hu2 = jnp.maximum(jnp.where(lane > c + 1, pc2 + qtc, ptc + qc2), 0.0)
            out_ref[c:c + 2, :] = jnp.where(lane2 == c + row2, 0.0, tail(hu1, hu2))


def kernel(adj_matrix, emb, W1, b1, W2, b2, W3, b3, Wmu, bmu, Wlv, blv,
           We1, be1, We2, be2, We3, be3):
    f32 = jnp.float32
    mu, lv, pt, qt = pl.pallas_call(
        _encoder_kernel,
        out_shape=[
            jax.ShapeDtypeStruct((_N, _LAT), f32),
            jax.ShapeDtypeStruct((_N, _LAT), f32),
            jax.ShapeDtypeStruct((_DH, _N), f32),
            jax.ShapeDtypeStruct((_DH, _N), f32),
        ],
    )(adj_matrix, emb,
      W1, b1.reshape(1, _HID), W2, b2.reshape(1, _HID), W3, b3.reshape(1, _HID),
      Wmu, bmu.reshape(1, _LAT), Wlv, blv.reshape(1, _LAT),
      We1[:_LAT], We1[_LAT:], be1.reshape(_DH, 1))

    w2t = We2.T
    zz = jnp.zeros((_HID, _DH), f32)
    w2bd = jnp.concatenate(
        [jnp.concatenate([w2t, zz], axis=1), jnp.concatenate([zz, w2t], axis=1)],
        axis=0)
    be2bd = jnp.tile(be2, 2).reshape(_DH, 1)
    we3bd = jnp.tile(We3[:, 0], 2).reshape(_DH, 1)
    be3r = be3.reshape(1, 1)

    adj_pred = pl.pallas_call(
        _decoder_kernel,
        grid=(_N // _RBLK, _N // _TBLK),
        in_specs=[
            pl.BlockSpec((_DH, _RBLK), lambda t, c: (0, t)),
            pl.BlockSpec((_DH, _RBLK), lambda t, c: (0, t)),
            pl.BlockSpec((_DH, _TBLK), lambda t, c: (0, c)),
            pl.BlockSpec((_DH, _TBLK), lambda t, c: (0, c)),
            pl.BlockSpec((_DH, 2 * _DH), lambda t, c: (0, 0)),
            pl.BlockSpec((_DH, 1), lambda t, c: (0, 0)),
            pl.BlockSpec((_DH, 1), lambda t, c: (0, 0)),
            pl.BlockSpec((1, 1), lambda t, c: (0, 0)),
        ],
        out_specs=pl.BlockSpec((_RBLK, _TBLK), lambda t, c: (t, c)),
        out_shape=jax.ShapeDtypeStruct((_N, _N), f32),
    )(pt, qt, pt, qt, w2bd, be2bd, we3bd, be3r)
    return adj_pred, mu, lv


# speedup vs baseline: 1.2917x; 1.2917x over previous
import jax
import jax.numpy as jnp
from jax.experimental import pallas as pl

_N = 1024
_FEAT = 8
_HID = 32
_LAT = 16
_DH = 2 * _HID
_RBLK = 128
_TBLK = 128


def _encoder_kernel(a_ref, x_ref, w1_ref, b1_ref, w2_ref, b2_ref, w3_ref, b3_ref,
                    wmu_ref, bmu_ref, wlv_ref, blv_ref, we1a_ref, we1b_ref, be1_ref,
                    mu_ref, lv_ref, pt_ref, qt_ref):
    f32 = jnp.float32
    a = a_ref[...]
    ii = jax.lax.broadcasted_iota(jnp.int32, (_N, _N), 0)
    jj = jax.lax.broadcasted_iota(jnp.int32, (_N, _N), 1)
    eye = ii == jj
    d_col = jnp.sum(jnp.where(eye, a, 0.0), axis=1, keepdims=True)
    a_hat = a + jnp.where(eye & (d_col == 0.0), 1.0, 0.0)
    ones = jnp.ones((_N, 1), f32)
    deg = jax.lax.dot_general(a_hat, ones, (((0,), (0,)), ((), ())),
                              preferred_element_type=f32)
    dis = jnp.where(deg > 0.0, jax.lax.rsqrt(deg), 0.0)

    def agg(v):
        return dis * jax.lax.dot_general(a_hat, dis * v, (((0,), (0,)), ((), ())),
                                         preferred_element_type=f32)

    x = x_ref[...]
    h1 = jnp.maximum(agg(jnp.dot(x, w1_ref[...], preferred_element_type=f32)) + b1_ref[...], 0.0)
    h2 = jnp.maximum(agg(jnp.dot(h1, w2_ref[...], preferred_element_type=f32)) + b2_ref[...], 0.0) + h1
    h3 = jnp.maximum(agg(jnp.dot(h2, w3_ref[...], preferred_element_type=f32)) + b3_ref[...], 0.0) + h2
    mu = jnp.dot(h3, wmu_ref[...], preferred_element_type=f32) + bmu_ref[...]
    mu_ref[...] = mu
    lv_ref[...] = jnp.dot(h3, wlv_ref[...], preferred_element_type=f32) + blv_ref[...]
    pt_ref[...] = jax.lax.dot_general(we1a_ref[...], mu, (((0,), (1,)), ((), ())),
                                      preferred_element_type=f32) + be1_ref[...]
    qt_ref[...] = jax.lax.dot_general(we1b_ref[...], mu, (((0,), (1,)), ((), ())),
                                      preferred_element_type=f32)


def _decoder_kernel(ptf_ref, qtf_ref, ptb_ref, qtb_ref, w2bd_ref, be2_ref,
                    we3_ref, be3_ref, out_ref):
    base = pl.program_id(0) * _RBLK
    ptf = ptf_ref[...]
    qtf = qtf_ref[...]
    w2bd = w2bd_ref[...]
    be2 = be2_ref[...]
    we3 = we3_ref[...]
    be3 = be3_ref[0, 0]
    lane = jax.lax.broadcasted_iota(jnp.int32, (1, _N), 1)
    lane2 = jax.lax.broadcasted_iota(jnp.int32, (2, _N), 1)
    row2 = jax.lax.broadcasted_iota(jnp.int32, (2, _N), 0)
    for s in range(_RBLK // 2):
        c = 2 * s
        i1 = base + c
        pc1 = ptb_ref[:, c:c + 1]
        qc1 = qtb_ref[:, c:c + 1]
        pc2 = ptb_ref[:, c + 1:c + 2]
        qc2 = qtb_ref[:, c + 1:c + 2]
        m = lane > i1
        hu1 = jnp.maximum(jnp.where(m, pc1 + qtf, ptf + qc1), 0.0)
        hu2 = jnp.maximum(jnp.where(m, pc2 + qtf, ptf + qc2), 0.0)
        hh = jnp.concatenate([hu1, hu2], axis=0).astype(jnp.bfloat16)
        g = jnp.maximum(jnp.dot(w2bd, hh, preferred_element_type=jnp.float32) + be2, 0.0)
        t = g * we3
        s1 = jnp.sum(t[0:_HID], axis=0, keepdims=True) + be3
        s2 = jnp.sum(t[_HID:], axis=0, keepdims=True) + be3
        s12 = jnp.concatenate([s1, s2], axis=0)
        out_ref[c:c + 2, :] = jnp.where(lane2 == i1 + row2, 0.0,
                                        jax.nn.sigmoid(s12))


def kernel(adj_matrix, emb, W1, b1, W2, b2, W3, b3, Wmu, bmu, Wlv, blv,
           We1, be1, We2, be2, We3, be3):
    f32 = jnp.float32
    mu, lv, pt, qt = pl.pallas_call(
        _encoder_kernel,
        out_shape=[
            jax.ShapeDtypeStruct((_N, _LAT), f32),
            jax.ShapeDtypeStruct((_N, _LAT), f32),
            jax.ShapeDtypeStruct((_DH, _N), f32),
            jax.ShapeDtypeStruct((_DH, _N), f32),
        ],
    )(adj_matrix, emb,
      W1, b1.reshape(1, _HID), W2, b2.reshape(1, _HID), W3, b3.reshape(1, _HID),
      Wmu, bmu.reshape(1, _LAT), Wlv, blv.reshape(1, _LAT),
      We1[:_LAT], We1[_LAT:], be1.reshape(_DH, 1))

    w2t = We2.T
    zz = jnp.zeros((_HID, _DH), f32)
    w2bd = jnp.concatenate(
        [jnp.concatenate([w2t, zz], axis=1), jnp.concatenate([zz, w2t], axis=1)],
        axis=0)
    be2bd = jnp.tile(be2, 2).reshape(_DH, 1)
    we3bd = jnp.tile(We3[:, 0], 2).reshape(_DH, 1)
    be3r = be3.reshape(1, 1)

    adj_pred = pl.pallas_call(
        _decoder_kernel,
        grid=(_N // _RBLK,),
        in_specs=[
            pl.BlockSpec((_DH, _N), lambda t: (0, 0)),
            pl.BlockSpec((_DH, _N), lambda t: (0, 0)),
            pl.BlockSpec((_DH, _RBLK), lambda t: (0, t)),
            pl.BlockSpec((_DH, _RBLK), lambda t: (0, t)),
            pl.BlockSpec((_DH, 2 * _DH), lambda t: (0, 0)),
            pl.BlockSpec((_DH, 1), lambda t: (0, 0)),
            pl.BlockSpec((_DH, 1), lambda t: (0, 0)),
            pl.BlockSpec((1, 1), lambda t: (0, 0)),
        ],
        out_specs=pl.BlockSpec((_RBLK, _N), lambda t: (t, 0)),
        out_shape=jax.ShapeDtypeStruct((_N, _N), f32),
    )(pt, qt, pt, qt, w2bd.astype(jnp.bfloat16), be2bd, we3bd, be3r)
    return adj_pred, mu, lv


# speedup vs baseline: 1.3852x; 1.0724x over previous
import jax
import jax.numpy as jnp
from jax.experimental import pallas as pl

_N = 1024
_FEAT = 8
_HID = 32
_LAT = 16
_DH = 2 * _HID
_RBLK = 128
_TBLK = 128


def _encoder_kernel(a_ref, x_ref, w1_ref, b1_ref, w2_ref, b2_ref, w3_ref, b3_ref,
                    wmu_ref, bmu_ref, wlv_ref, blv_ref, we1a_ref, we1b_ref, be1_ref,
                    mu_ref, lv_ref, pt_ref, qt_ref):
    f32 = jnp.float32
    a = a_ref[...]
    ii = jax.lax.broadcasted_iota(jnp.int32, (_N, _N), 0)
    jj = jax.lax.broadcasted_iota(jnp.int32, (_N, _N), 1)
    eye = ii == jj
    d_col = jnp.sum(jnp.where(eye, a, 0.0), axis=1, keepdims=True)
    a_hat = a + jnp.where(eye & (d_col == 0.0), 1.0, 0.0)
    ones = jnp.ones((_N, 1), f32)
    deg = jax.lax.dot_general(a_hat, ones, (((0,), (0,)), ((), ())),
                              preferred_element_type=f32)
    dis = jnp.where(deg > 0.0, jax.lax.rsqrt(deg), 0.0)

    def agg(v):
        return dis * jax.lax.dot_general(a_hat, dis * v, (((0,), (0,)), ((), ())),
                                         preferred_element_type=f32)

    x = x_ref[...]
    h1 = jnp.maximum(agg(jnp.dot(x, w1_ref[...], preferred_element_type=f32)) + b1_ref[...], 0.0)
    h2 = jnp.maximum(agg(jnp.dot(h1, w2_ref[...], preferred_element_type=f32)) + b2_ref[...], 0.0) + h1
    h3 = jnp.maximum(agg(jnp.dot(h2, w3_ref[...], preferred_element_type=f32)) + b3_ref[...], 0.0) + h2
    mu = jnp.dot(h3, wmu_ref[...], preferred_element_type=f32) + bmu_ref[...]
    mu_ref[...] = mu
    lv_ref[...] = jnp.dot(h3, wlv_ref[...], preferred_element_type=f32) + blv_ref[...]
    pt_ref[...] = (jax.lax.dot_general(we1a_ref[...], mu, (((0,), (1,)), ((), ())),
                                       preferred_element_type=f32)
                   + be1_ref[...]).astype(jnp.bfloat16)
    qt_ref[...] = jax.lax.dot_general(we1b_ref[...], mu, (((0,), (1,)), ((), ())),
                                      preferred_element_type=f32).astype(jnp.bfloat16)


def _decoder_kernel(ptf_ref, qtf_ref, ptb_ref, qtb_ref, w2bd_ref, be2_ref,
                    we3_ref, be3_ref, out_ref):
    base = pl.program_id(0) * _RBLK
    ptf = ptf_ref[...]
    qtf = qtf_ref[...]
    w2bd = w2bd_ref[...]
    be2 = be2_ref[...]
    we3 = we3_ref[...]
    be3 = be3_ref[0, 0]
    lane16 = jax.lax.broadcasted_iota(jnp.int32, (1, _N), 1).astype(jnp.int16)
    lane2 = jax.lax.broadcasted_iota(jnp.int32, (2, _N), 1)
    row2 = jax.lax.broadcasted_iota(jnp.int32, (2, _N), 0)
    zero16 = jnp.zeros((), jnp.bfloat16)
    for s in range(_RBLK // 2):
        c = 2 * s
        i1 = base + c
        pc1 = ptb_ref[:, c:c + 1]
        qc1 = qtb_ref[:, c:c + 1]
        pc2 = ptb_ref[:, c + 1:c + 2]
        qc2 = qtb_ref[:, c + 1:c + 2]
        m = lane16 > i1.astype(jnp.int16)
        hu1 = jnp.maximum(jnp.where(m, pc1 + qtf, ptf + qc1), zero16)
        hu2 = jnp.maximum(jnp.where(m, pc2 + qtf, ptf + qc2), zero16)
        hh = jnp.concatenate([hu1, hu2], axis=0)
        g = jnp.maximum(jnp.dot(w2bd, hh, preferred_element_type=jnp.float32) + be2, 0.0)
        t = g * we3
        s1 = jnp.sum(t[0:_HID], axis=0, keepdims=True) + be3
        s2 = jnp.sum(t[_HID:], axis=0, keepdims=True) + be3
        s12 = jnp.concatenate([s1, s2], axis=0)
        out_ref[c:c + 2, :] = jnp.where(lane2 == i1 + row2, 0.0,
                                        jax.nn.sigmoid(s12))


def kernel(adj_matrix, emb, W1, b1, W2, b2, W3, b3, Wmu, bmu, Wlv, blv,
           We1, be1, We2, be2, We3, be3):
    f32 = jnp.float32
    mu, lv, pt, qt = pl.pallas_call(
        _encoder_kernel,
        out_shape=[
            jax.ShapeDtypeStruct((_N, _LAT), f32),
            jax.ShapeDtypeStruct((_N, _LAT), f32),
            jax.ShapeDtypeStruct((_DH, _N), jnp.bfloat16),
            jax.ShapeDtypeStruct((_DH, _N), jnp.bfloat16),
        ],
    )(adj_matrix, emb,
      W1, b1.reshape(1, _HID), W2, b2.reshape(1, _HID), W3, b3.reshape(1, _HID),
      Wmu, bmu.reshape(1, _LAT), Wlv, blv.reshape(1, _LAT),
      We1[:_LAT], We1[_LAT:], be1.reshape(_DH, 1))

    w2t = We2.T
    zz = jnp.zeros((_HID, _DH), f32)
    w2bd = jnp.concatenate(
        [jnp.concatenate([w2t, zz], axis=1), jnp.concatenate([zz, w2t], axis=1)],
        axis=0)
    be2bd = jnp.tile(be2, 2).reshape(_DH, 1)
    we3bd = jnp.tile(We3[:, 0], 2).reshape(_DH, 1)
    be3r = be3.reshape(1, 1)

    adj_pred = pl.pallas_call(
        _decoder_kernel,
        grid=(_N // _RBLK,),
        in_specs=[
            pl.BlockSpec((_DH, _N), lambda t: (0, 0)),
            pl.BlockSpec((_DH, _N), lambda t: (0, 0)),
            pl.BlockSpec((_DH, _RBLK), lambda t: (0, t)),
            pl.BlockSpec((_DH, _RBLK), lambda t: (0, t)),
            pl.BlockSpec((_DH, 2 * _DH), lambda t: (0, 0)),
            pl.BlockSpec((_DH, 1), lambda t: (0, 0)),
            pl.BlockSpec((_DH, 1), lambda t: (0, 0)),
            pl.BlockSpec((1, 1), lambda t: (0, 0)),
        ],
        out_specs=pl.BlockSpec((_RBLK, _N), lambda t: (t, 0)),
        out_shape=jax.ShapeDtypeStruct((_N, _N), f32),
    )(pt, qt, pt, qt, w2bd.astype(jnp.bfloat16), be2bd, we3bd, be3r)
    return adj_pred, mu, lv
